# baseline (speedup 1.0000x reference)
import jax
import jax.numpy as jnp
from jax import lax
from jax.experimental import pallas as pl
from jax.experimental.pallas import tpu as pltpu
from jax.experimental.pallas import tpu_sc as plsc

D = 128
E = 160000
NJ = 10000
C1 = 256
BN_EPS = 1e-5

NC, NS = 2, 16
NW = NC * NS

EW_G = (2 * E) // NW
KG = 80
NCG = EW_G // KG
NJP = 10240

BE = 1280
NBLK = E // BE

_sc_mesh = plsc.VectorSubcoreMesh(
    core_axis_name="c", subcore_axis_name="s", num_cores=NC, num_subcores=NS
)


def _gather_body(t_hbm, idx_hbm, idxf_hbm, g_hbm, gf_hbm,
                 idx_c, idxf_c, rows_v, rows2_v, sem):
    wid = lax.axis_index("s") * NC + lax.axis_index("c")

    def chunk(c, carry):
        b = pl.multiple_of(wid * EW_G + c * KG, 8)
        pltpu.sync_copy(idx_hbm.at[pl.ds(b, KG)], idx_c)
        pltpu.sync_copy(idxf_hbm.at[pl.ds(b, KG)], idxf_c)
        pltpu.async_copy(t_hbm.at[idx_c], rows_v, sem).wait()
        pltpu.sync_copy(rows_v, g_hbm.at[pl.ds(b, KG)])
        pltpu.async_copy(t_hbm.at[idxf_c], rows2_v, sem).wait()
        pltpu.sync_copy(rows2_v, gf_hbm.at[pl.ds(b, KG)])
        return carry

    lax.fori_loop(0, NCG, chunk, 0)


_gather = pl.kernel(
    _gather_body,
    out_type=(
        jax.ShapeDtypeStruct((2 * E, D), jnp.float32),
        jax.ShapeDtypeStruct((2 * E, D), jnp.float32),
    ),
    mesh=_sc_mesh,
    scratch_types=[
        pltpu.VMEM((KG,), jnp.int32),
        pltpu.VMEM((KG,), jnp.int32),
        pltpu.VMEM((KG, D), jnp.float32),
        pltpu.VMEM((KG, D), jnp.float32),
        pltpu.SemaphoreType.DMA,
    ],
)


def _make_gather1(m_rows, k):
    ew = m_rows // NW
    nch = ew // k

    def body(t_hbm, idx_hbm, out_hbm, idx_c, rows_v, sem):
        wid = lax.axis_index("s") * NC + lax.axis_index("c")

        def chunk(c, carry):
            b = pl.multiple_of(wid * ew + c * k, 8)
            pltpu.sync_copy(idx_hbm.at[pl.ds(b, k)], idx_c)
            pltpu.async_copy(t_hbm.at[idx_c], rows_v, sem).wait()
            pltpu.sync_copy(rows_v, out_hbm.at[pl.ds(b, k)])
            return carry

        lax.fori_loop(0, nch, chunk, 0)

    return pl.kernel(
        body,
        out_type=jax.ShapeDtypeStruct((m_rows, D), jnp.float32),
        mesh=_sc_mesh,
        scratch_types=[
            pltpu.VMEM((k,), jnp.int32),
            pltpu.VMEM((k, D), jnp.float32),
            pltpu.SemaphoreType.DMA,
        ],
    )


_gather_perm = _make_gather1(E, 40)
_gather_bounds = _make_gather1(NJP, 40)


def _passA_body(g_ref, gf_ref, enc_ref, wa_ref, wb_ref, wc_ref, y_ref, st_ref):
    gb = g_ref[...].astype(jnp.bfloat16)
    gfb = gf_ref[...].astype(jnp.bfloat16)
    eb = enc_ref[...].astype(jnp.bfloat16)
    y = lax.dot_general(wa_ref[...], gb, (((1,), (1,)), ((), ())),
                        preferred_element_type=jnp.float32)
    y = y + lax.dot_general(wb_ref[...], gfb, (((1,), (1,)), ((), ())),
                            preferred_element_type=jnp.float32)
    y = y + lax.dot_general(wc_ref[...], eb, (((1,), (0,)), ((), ())),
                            preferred_element_type=jnp.float32)
    y_ref[...] = y.astype(jnp.bfloat16)

    @pl.when(pl.program_id(0) == 0)
    def _():
        st_ref[...] = jnp.zeros_like(st_ref)

    s = jnp.zeros((C1, 128), jnp.float32)
    q = jnp.zeros((C1, 128), jnp.float32)
    for k in range(BE // 128):
        blk = y[:, k * 128:(k + 1) * 128]
        s = s + blk
        q = q + blk * blk
    st_ref[0] += s
    st_ref[1] += q


def _make_passA(stream):
    return pl.pallas_call(
        _passA_body,
        grid=(NBLK,),
        in_specs=[
            pl.BlockSpec((BE, D), lambda i: (i + stream * NBLK, 0)),
            pl.BlockSpec((BE, D), lambda i: (i + stream * NBLK, 0)),
            pl.BlockSpec((D, BE), lambda i: (0, i)),
            pl.BlockSpec((C1, D), lambda i: (0, 0)),
            pl.BlockSpec((C1, D), lambda i: (0, 0)),
            pl.BlockSpec((C1, D), lambda i: (0, 0)),
        ],
        out_specs=[
            pl.BlockSpec((C1, BE), lambda i: (0, i)),
            pl.BlockSpec((2, C1, 128), lambda i: (0, 0, 0)),
        ],
        out_shape=(
            jax.ShapeDtypeStruct((C1, E), jnp.bfloat16),
            jax.ShapeDtypeStruct((2, C1, 128), jnp.float32),
        ),
    )


def _passB_body(y_ref, st_ref, bnw_ref, bnb_ref, w2_ref, b2_ref, z_ref):
    st = st_ref[...]
    s = jnp.sum(st[0], axis=1, keepdims=True)
    q = jnp.sum(st[1], axis=1, keepdims=True)
    mean = s * (1.0 / E)
    var = q * (1.0 / E) - mean * mean
    inv = lax.rsqrt(var + BN_EPS)
    a = bnw_ref[:, 0:1] * inv
    c = bnb_ref[:, 0:1] - mean * a
    y = y_ref[...].astype(jnp.float32)
    h = jnp.maximum(y * a + c, 0.0).astype(jnp.bfloat16)
    z = lax.dot_general(h, w2_ref[...], (((0,), (1,)), ((), ())),
                        preferred_element_type=jnp.float32)
    z_ref[...] = z + b2_ref[...]


_passB = pl.pallas_call(
    _passB_body,
    grid=(NBLK,),
    in_specs=[
        pl.BlockSpec((C1, BE), lambda i: (0, i)),
        pl.BlockSpec((2, C1, 128), lambda i: (0, 0, 0)),
        pl.BlockSpec((C1, 128), lambda i: (0, 0)),
        pl.BlockSpec((C1, 128), lambda i: (0, 0)),
        pl.BlockSpec((D, C1), lambda i: (0, 0)),
        pl.BlockSpec((1, D), lambda i: (0, 0)),
    ],
    out_specs=pl.BlockSpec((BE, D), lambda i: (i, 0)),
    out_shape=jax.ShapeDtypeStruct((E, D), jnp.float32),
)


def _cumsum_body(zs_ref, c_ref, carry_ref):
    @pl.when(pl.program_id(0) == 0)
    def _():
        carry_ref[...] = jnp.zeros_like(carry_ref)

    x = zs_ref[...]
    s = 1
    while s < BE:
        x = x + jnp.concatenate(
            [jnp.zeros((s, D), x.dtype), x[:BE - s]], axis=0)
        s *= 2
    c = x + carry_ref[0:1, :]
    c_ref[...] = c
    carry_ref[0:1, :] = c[BE - 1:BE, :]


_cumsum = pl.pallas_call(
    _cumsum_body,
    grid=(NBLK,),
    in_specs=[pl.BlockSpec((BE, D), lambda i: (i, 0))],
    out_specs=pl.BlockSpec((BE, D), lambda i: (i, 0)),
    out_shape=jax.ShapeDtypeStruct((E, D), jnp.float32),
    scratch_shapes=[pltpu.VMEM((8, D), jnp.float32)],
)


def _combine_body(ld_ref, s_ref, valid_ref, cnt_ref, out_ref):
    t = s_ref[...] * valid_ref[:, 0:1]
    head = t[0:1, :] * jnp.zeros((1, 1), jnp.float32)
    tm1 = jnp.concatenate([head, t[:NJ - 1]], axis=0)
    ssum = t[:NJ] - tm1
    upd = ssum / jnp.maximum(cnt_ref[:NJ, 0:1], 1.0)
    r = lax.broadcasted_iota(jnp.int32, (D, D), 0)
    c = lax.broadcasted_iota(jnp.int32, (D, D), 1)
    eye = (r == c).astype(jnp.float32)
    upd_t = lax.dot_general(eye, upd, (((1,), (1,)), ((), ())),
                            preferred_element_type=jnp.float32,
                            precision=lax.Precision.HIGHEST)
    out_ref[...] = ld_ref[...] + upd_t[None, :, :]


_combine = pl.pallas_call(
    _combine_body,
    grid=(1,),
    in_specs=[
        pl.BlockSpec((1, D, NJ), lambda i: (0, 0, 0)),
        pl.BlockSpec((NJP, D), lambda i: (0, 0)),
        pl.BlockSpec((NJP, 16), lambda i: (0, 0)),
        pl.BlockSpec((NJP, 16), lambda i: (0, 0)),
    ],
    out_specs=pl.BlockSpec((1, D, NJ), lambda i: (0, 0, 0)),
    out_shape=jax.ShapeDtypeStruct((1, D, NJ), jnp.float32),
)


def kernel(ldesc0, ldesc1, line_enc0, line_enc1,
           lines_junc_idx0, lines_junc_idx1, W1, b1, bn_w, bn_b, W2, b2):
    del b1
    f32 = jnp.float32

    table = jnp.concatenate(
        [jnp.transpose(ldesc0[0]), jnp.transpose(ldesc1[0])], axis=0)
    i0 = lines_junc_idx0[0]
    i1 = lines_junc_idx1[0]
    i0f = i0.reshape(-1, 2)[:, ::-1].reshape(-1)
    i1f = i1.reshape(-1, 2)[:, ::-1].reshape(-1)
    idx_all = jnp.concatenate([i0, i1 + NJ])
    idxf_all = jnp.concatenate([i0f, i1f + NJ])

    G, Gf = _gather(table, idx_all, idxf_all)

    W1h = W1.astype(jnp.bfloat16)
    Wa, Wb, Wc = W1h[:, :D], W1h[:, D:2 * D], W1h[:, 2 * D:]
    bnw2 = jnp.broadcast_to(bn_w[:, None], (C1, 128))
    bnb2 = jnp.broadcast_to(bn_b[:, None], (C1, 128))
    W2h = W2.astype(jnp.bfloat16)
    b2r = b2[None, :]

    outs = []
    for stream, (ld, enc, ii) in ((0, (ldesc0, line_enc0, i0)),
                                  (1, (ldesc1, line_enc1, i1))):
        y, st = _make_passA(stream)(G, Gf, enc[0], Wa, Wb, Wc)
        z = _passB(y, st, bnw2, bnb2, W2h, b2r)

        perm = jnp.argsort(ii).astype(jnp.int32)
        sidx = jnp.take(ii, perm)
        end = jnp.searchsorted(sidx, jnp.arange(NJ, dtype=jnp.int32),
                               side='right').astype(jnp.int32)
        start = jnp.concatenate([jnp.zeros((1,), jnp.int32), end[:-1]])
        cnt = (end - start).astype(f32)
        bidx = jnp.maximum(end - 1, 0)
        valid = (end > 0).astype(f32)
        pad = jnp.zeros((NJP - NJ,), jnp.int32)
        bidx_pad = jnp.concatenate([bidx, pad])
        valid16 = jnp.broadcast_to(
            jnp.concatenate([valid, pad.astype(f32)])[:, None], (NJP, 16))
        cnt16 = jnp.broadcast_to(
            jnp.concatenate([cnt, pad.astype(f32)])[:, None], (NJP, 16))

        zs = _gather_perm(z, perm)
        csum = _cumsum(zs)
        bounds = _gather_bounds(csum, bidx_pad)
        outs.append(_combine(ld, bounds, valid16, cnt16))

    return (outs[0], outs[1])
